# baseline (speedup 1.0000x reference)
import functools

import jax
import jax.numpy as jnp
from jax import lax
from jax.experimental import pallas as pl
from jax.experimental.pallas import tpu as pltpu
from jax.experimental.pallas import tpu_sc as plsc

E = 320000
N = 10000
S = 30
D = 128

_NC = 2
_NS = 16
_NW = _NC * _NS

_SP = 32
_NPT = 320
_NPAD = _NW * _NPT
_NPC = 128 // _SP
_CH = _NPT // _NPC
_K = 4


_RB = 20


def _attn_body(zf_ref, zo_ref, w1_ref, w2_ref, u_ref):
    zf = zf_ref[...]
    zo = zo_ref[...]
    w1 = w1_ref[...][0]
    w2 = w2_ref[...][0]
    x = jnp.sum(zf * w1, axis=3) + jnp.sum(zo * w2, axis=3)
    u_ref[...] = 1.0 + jnp.exp(x)


def _edge_weights(z_feature, z_others, W_attn):
    gr = E // (128 * _RB)
    zf4 = z_feature.reshape(gr, _RB, 128, D)
    zo4 = z_others.reshape(gr, _RB, 128, D)
    w1 = W_attn[:D, 0].reshape(1, D)
    w2 = W_attn[D:, 0].reshape(1, D)
    u = pl.pallas_call(
        _attn_body,
        grid=(gr,),
        in_specs=[
            pl.BlockSpec((1, _RB, 128, D), lambda i: (i, 0, 0, 0)),
            pl.BlockSpec((1, _RB, 128, D), lambda i: (i, 0, 0, 0)),
            pl.BlockSpec((1, D), lambda i: (0, 0)),
            pl.BlockSpec((1, D), lambda i: (0, 0)),
        ],
        out_specs=pl.BlockSpec((1, _RB, 128), lambda i: (i, 0, 0)),
        out_shape=jax.ShapeDtypeStruct((gr, _RB, 128), jnp.float32),
    )(zf4, zo4, w1, w2)
    return u.reshape(E)


def _sc_body(u_hbm, z_hbm, idxz_hbm, idxu_hbm, out_hbm,
             idxz, idxu, ugv, zbuf, obuf, sem, osem):
    wid = lax.axis_index("s") * _NC + lax.axis_index("c")

    pltpu.sync_copy(idxz_hbm.at[wid], idxz)
    pltpu.sync_copy(idxu_hbm.at[wid], idxu)

    def gstart(c, b):
        pltpu.make_async_copy(z_hbm.at[idxz.at[c]], zbuf.at[b], sem.at[b]).start()
        pltpu.make_async_copy(u_hbm.at[idxu.at[c]], ugv.at[c], sem.at[b]).start()

    def gwait(c, b):
        pltpu.make_async_copy(z_hbm.at[idxz.at[c]], zbuf.at[b], sem.at[b]).wait()
        pltpu.make_async_copy(u_hbm.at[idxu.at[c]], ugv.at[c], sem.at[b]).wait()

    def odesc(row0, w):
        dst = out_hbm.at[pl.ds(wid * _NPT + row0, 2 * _NPC)]
        return pltpu.make_async_copy(obuf.at[w], dst, osem.at[w])

    for p in range(_K - 1):
        gstart(p, p)

    @pl.loop(0, _CH, step=_K)
    def _main(cc):
        for b in range(_K):
            c = cc + b

            @pl.when(c + _K - 1 < _CH)
            def _():
                gstart(c + _K - 1, (b - 1) % _K)

            gwait(c, b)

            half = b % 2
            w = b // 2

            if half == 0:
                @pl.when(c >= 4)
                def _():
                    odesc((c - 4) * _NPC, w).wait()

            c_vec = jnp.full((16,), c, jnp.int32)
            last = jnp.full((16,), 15, jnp.int32)
            for q in range(_NPC):
                s0 = _SP * q
                t0 = ugv[c, pl.ds(s0, 16)]
                t1 = ugv[c, pl.ds(s0 + 16, 16)]
                cs = plsc.cumsum(t0 + t1)
                tot = lax.gather(
                    cs,
                    last[:, None],
                    lax.GatherDimensionNumbers(
                        offset_dims=(),
                        collapsed_slice_dims=(0,),
                        start_index_map=(0,),
                    ),
                    slice_sizes=(1,),
                    mode=lax.GatherScatterMode.PROMISE_IN_BOUNDS,
                )
                r = 1.0 / jnp.where(tot > 0.0, tot, 1.0)
                acc = [jnp.zeros((16,), jnp.float32) for _ in range(8)]
                for s in range(_SP):
                    wv = plsc.load_gather(
                        ugv, [c_vec, jnp.full((16,), s0 + s, jnp.int32)]
                    )
                    row = s0 + s
                    for k in range(8):
                        acc[k] = acc[k] + wv * zbuf[b, row, pl.ds(16 * k, 16)]
                for k in range(8):
                    obuf[w, half * _NPC + q, pl.ds(16 * k, 16)] = acc[k] * r

            if half == 1:
                odesc((c - 1) * _NPC, w).start()

    odesc((_CH - 4) * _NPC, 0).wait()
    odesc((_CH - 2) * _NPC, 1).wait()


@functools.partial(
    pl.kernel,
    out_type=jax.ShapeDtypeStruct((_NPAD, D), jnp.float32),
    mesh=plsc.VectorSubcoreMesh(core_axis_name="c", subcore_axis_name="s"),
    compiler_params=pltpu.CompilerParams(needs_layout_passes=False),
    scratch_types=[
        pltpu.VMEM((_CH, 128), jnp.int32),
        pltpu.VMEM((_CH, 128), jnp.int32),
        pltpu.VMEM((_CH, 128), jnp.float32),
        pltpu.VMEM((_K, 128, D), jnp.float32),
        pltpu.VMEM((2, 2 * _NPC, D), jnp.float32),
        pltpu.SemaphoreType.DMA((_K,)),
        pltpu.SemaphoreType.DMA((2,)),
    ],
)
def _sc_reduce(u_hbm, z_hbm, idxz_hbm, idxu_hbm, out_hbm,
               idxz, idxu, ugv, zbuf, obuf, sem, osem):
    _sc_body(u_hbm, z_hbm, idxz_hbm, idxu_hbm, out_hbm,
             idxz, idxu, ugv, zbuf, obuf, sem, osem)


def kernel(z_feature, z_others, scope, W_attn):
    u = _edge_weights(z_feature, z_others, W_attn)
    u_pad = jnp.concatenate([u, jnp.zeros((8,), jnp.float32)])

    sc = jnp.zeros((_NPAD, _SP), jnp.int32)
    sc = sc.at[:N, :S].set(scope.astype(jnp.int32))
    idxz = jnp.maximum(sc - 1, 0).reshape(_NW, _CH, 128)
    idxu = jnp.where(sc == 0, E, sc - 1).reshape(_NW, _CH, 128)

    out = _sc_reduce(u_pad, z_others, idxz, idxu)
    return jnp.concatenate([jnp.zeros((1, D), jnp.float32), out[:N]], axis=0)
